# baseline (speedup 1.0000x reference)
import functools

import jax
import jax.numpy as jnp
from jax import lax
from jax.experimental import pallas as pl
from jax.experimental.pallas import tpu as pltpu
from jax.experimental.pallas import tpu_sc as plsc

VOCAB_ = 1000000
EMBED = 64
B_ = 16384
C_ = 20

_NC = 2
_NS = 16
_NW = _NC * _NS
_BPW = B_ // _NW
_CB = 32
_NCHUNK = _BPW // _CB
_G = 4


def _sc_kernel(tgt_tab, ctx_tab, tidx_hbm, cidx_hbm, out_hbm,
               tcid_v, trows_v, cidx_v, crows_v, out_v, accmat_v, sem):
    wid = lax.axis_index("s") * _NC + lax.axis_index("c")
    base = wid * _BPW
    lane = lax.broadcasted_iota(jnp.int32, (16,), 0)
    sidx = lane * (_G * C_)

    def chunk_body(i, _):
        flat = (base + i * _CB) * C_
        pltpu.sync_copy(tidx_hbm.at[pl.ds(base + i * _CB, _CB)],
                        tcid_v.at[pl.ds(0, _CB)])
        pltpu.sync_copy(cidx_hbm.at[pl.ds(flat, _CB * C_)],
                        cidx_v.at[pl.ds(0, _CB * C_)])
        tcid_v[pl.ds(_CB, 16)] = jnp.right_shift(tcid_v[pl.ds(0, 16)], 1)
        tcid_v[pl.ds(_CB + 16, 16)] = jnp.right_shift(tcid_v[pl.ds(16, 16)], 1)
        cp = pltpu.async_copy(
            tgt_tab.at[tcid_v.at[pl.ds(_CB, _CB)]], trows_v, sem)
        for k in range(_CB * C_ // 16):
            cidx_v[pl.ds(_CB * C_ + 16 * k, 16)] = jnp.right_shift(
                cidx_v[pl.ds(16 * k, 16)], 1)
        cp.wait()
        pltpu.async_copy(
            ctx_tab.at[cidx_v.at[pl.ds(_CB * C_, _CB * C_)]],
            crows_v, sem).wait()

        def group_body(g, _):
            tv = tcid_v[pl.ds(g * _G, 16)]
            for j in range(_G):
                row = g * _G + j
                toff = (tv[j] & 1) * EMBED
                t = [trows_v[row, pl.ds(toff + 16 * m, 16)] for m in range(4)]
                civ = None
                for c in range(C_):
                    rl = j * C_ + c
                    r = g * (_G * C_) + rl
                    k, ln = divmod(rl, 16)
                    if ln == 0 or civ is None:
                        civ = cidx_v[pl.ds(g * (_G * C_) + k * 16, 16)]
                    coff = (civ[ln] & 1) * EMBED
                    acc = crows_v[r, pl.ds(coff, 16)] * t[0]
                    acc += crows_v[r, pl.ds(coff + 16, 16)] * t[1]
                    acc += crows_v[r, pl.ds(coff + 32, 16)] * t[2]
                    acc += crows_v[r, pl.ds(coff + 48, 16)] * t[3]
                    plsc.store_scatter(accmat_v, [sidx + rl], acc)
            for k in range(_G * C_ // 16):
                s = accmat_v[pl.ds(16 * k, 16)]
                for m in range(1, 16):
                    s += accmat_v[pl.ds(m * _G * C_ + 16 * k, 16)]
                out_v[pl.ds(g * _G * C_ + 16 * k, 16)] = s
            return _

        lax.fori_loop(0, _CB // _G, group_body, None)
        pltpu.sync_copy(out_v, out_hbm.at[pl.ds(flat, _CB * C_)])
        return _

    lax.fori_loop(0, _NCHUNK, chunk_body, None)


@jax.jit
def _run(target_words, context_flat, tgt_pairs, ctx_pairs):
    mesh = plsc.VectorSubcoreMesh(core_axis_name="c", subcore_axis_name="s")
    k = functools.partial(
        pl.kernel,
        mesh=mesh,
        compiler_params=pltpu.CompilerParams(needs_layout_passes=False),
        out_type=jax.ShapeDtypeStruct((B_ * C_,), jnp.float32),
        scratch_types=[
            pltpu.VMEM((2 * _CB,), jnp.int32),
            pltpu.VMEM((_CB, 2 * EMBED), jnp.float32),
            pltpu.VMEM((2 * _CB * C_,), jnp.int32),
            pltpu.VMEM((_CB * C_, 2 * EMBED), jnp.float32),
            pltpu.VMEM((_CB * C_,), jnp.float32),
            pltpu.VMEM((16 * _G * C_,), jnp.float32),
            pltpu.SemaphoreType.DMA,
        ],
    )(_sc_kernel)
    return k(tgt_pairs, ctx_pairs, target_words, context_flat)


def kernel(target_words, context_words, target_table, context_table):
    context_flat = context_words.reshape(-1)
    tgt_pairs = target_table.reshape(VOCAB_ // 2, 2 * EMBED)
    ctx_pairs = context_table.reshape(VOCAB_ // 2, 2 * EMBED)
    return _run(target_words, context_flat, tgt_pairs, ctx_pairs).reshape(B_, C_)


# speedup vs baseline: 1.1908x; 1.1908x over previous
import functools

import jax
import jax.numpy as jnp
from jax import lax
from jax.experimental import pallas as pl
from jax.experimental.pallas import tpu as pltpu
from jax.experimental.pallas import tpu_sc as plsc

VOCAB_ = 1000000
EMBED = 64
B_ = 16384
C_ = 20

_NC = 2
_NS = 16
_NW = _NC * _NS
_BPW = B_ // _NW
_CB = 32
_NCHUNK = _BPW // _CB
_G = 4


def _sc_kernel(tgt_tab, ctx_tab, tidx_hbm, cidx_hbm, out_hbm,
               tcid_v, trows_v, cidx_v, crows_v, out_v, accmat_v, sem):
    wid = lax.axis_index("s") * _NC + lax.axis_index("c")
    base = wid * _BPW
    lane = lax.broadcasted_iota(jnp.int32, (16,), 0)
    sidx = lane * (_G * C_)

    def chunk_body(i, _):
        flat = (base + i * _CB) * C_
        pltpu.sync_copy(tidx_hbm.at[pl.ds(base + i * _CB, _CB)],
                        tcid_v.at[pl.ds(0, _CB)])
        pltpu.sync_copy(cidx_hbm.at[pl.ds(flat, _CB * C_)],
                        cidx_v.at[pl.ds(0, _CB * C_)])
        tcps = []
        for j in range(_CB):
            tv = tcid_v[pl.ds((j // 16) * 16, 16)]
            tcps.append(pltpu.async_copy(
                tgt_tab.at[tv[j % 16]], trows_v.at[j, pl.ds(0, EMBED)], sem))
        for k in range(_CB * C_ // 16):
            cidx_v[pl.ds(_CB * C_ + 16 * k, 16)] = jnp.right_shift(
                cidx_v[pl.ds(16 * k, 16)], 1)
        for cp in tcps:
            cp.wait()
        pltpu.async_copy(
            ctx_tab.at[cidx_v.at[pl.ds(_CB * C_, _CB * C_)]],
            crows_v, sem).wait()

        def group_body(g, _):
            for j in range(_G):
                row = g * _G + j
                t = [trows_v[row, pl.ds(16 * m, 16)] for m in range(4)]
                civ = None
                for c in range(C_):
                    rl = j * C_ + c
                    r = g * (_G * C_) + rl
                    k, ln = divmod(rl, 16)
                    if ln == 0 or civ is None:
                        civ = cidx_v[pl.ds(g * (_G * C_) + k * 16, 16)]
                    coff = (civ[ln] & 1) * EMBED
                    acc = crows_v[r, pl.ds(coff, 16)] * t[0]
                    acc += crows_v[r, pl.ds(coff + 16, 16)] * t[1]
                    acc += crows_v[r, pl.ds(coff + 32, 16)] * t[2]
                    acc += crows_v[r, pl.ds(coff + 48, 16)] * t[3]
                    plsc.store_scatter(accmat_v, [sidx + rl], acc)
            for k in range(_G * C_ // 16):
                s = accmat_v[pl.ds(16 * k, 16)]
                for m in range(1, 16):
                    s += accmat_v[pl.ds(m * _G * C_ + 16 * k, 16)]
                out_v[pl.ds(g * _G * C_ + 16 * k, 16)] = s
            return _

        lax.fori_loop(0, _CB // _G, group_body, None)
        pltpu.sync_copy(out_v, out_hbm.at[pl.ds(flat, _CB * C_)])
        return _

    lax.fori_loop(0, _NCHUNK, chunk_body, None)


@jax.jit
def _run(target_words, context_flat, tgt_tab, ctx_pairs):
    mesh = plsc.VectorSubcoreMesh(core_axis_name="c", subcore_axis_name="s")
    k = functools.partial(
        pl.kernel,
        mesh=mesh,
        compiler_params=pltpu.CompilerParams(needs_layout_passes=False),
        out_type=jax.ShapeDtypeStruct((B_ * C_,), jnp.float32),
        scratch_types=[
            pltpu.VMEM((2 * _CB,), jnp.int32),
            pltpu.VMEM((_CB, 2 * EMBED), jnp.float32),
            pltpu.VMEM((2 * _CB * C_,), jnp.int32),
            pltpu.VMEM((_CB * C_, 2 * EMBED), jnp.float32),
            pltpu.VMEM((_CB * C_,), jnp.float32),
            pltpu.VMEM((16 * _G * C_,), jnp.float32),
            pltpu.SemaphoreType.DMA,
        ],
    )(_sc_kernel)
    return k(tgt_tab, ctx_pairs, target_words, context_flat)


def kernel(target_words, context_words, target_table, context_table):
    context_flat = context_words.reshape(-1)
    ctx_pairs = context_table.reshape(VOCAB_ // 2, 2 * EMBED)
    return _run(target_words, context_flat, target_table, ctx_pairs).reshape(B_, C_)


# speedup vs baseline: 1.3332x; 1.1195x over previous
import functools

import jax
import jax.numpy as jnp
from jax import lax
from jax.experimental import pallas as pl
from jax.experimental.pallas import tpu as pltpu
from jax.experimental.pallas import tpu_sc as plsc

VOCAB_ = 1000000
EMBED = 64
B_ = 16384
C_ = 20

_NC = 2
_NS = 16
_NW = _NC * _NS
_BPW = B_ // _NW
_CB = 16
_NCHUNK = _BPW // _CB
_G = 4


def _sc_kernel(tgt_tab, ctx_tab, tidx_hbm, cidx_hbm, out_hbm,
               tcid_v, trows_v, cidx_v, crows_v, out_v, accmat_v, sem):
    wid = lax.axis_index("s") * _NC + lax.axis_index("c")
    base = wid * _BPW
    lane = lax.broadcasted_iota(jnp.int32, (16,), 0)
    sidx = lane * (_G * C_)

    def chunk_body(i, _):
        flat = (base + i * _CB) * C_
        pltpu.sync_copy(tidx_hbm.at[pl.ds(base + i * _CB, _CB)], tcid_v)
        pltpu.sync_copy(cidx_hbm.at[pl.ds(flat, _CB * C_)], cidx_v)
        cps = []
        tv = tcid_v[pl.ds(0, 16)]
        for j in range(_CB):
            cps.append(pltpu.async_copy(
                tgt_tab.at[tv[j]], trows_v.at[j], sem))
        for r in range(_CB * C_):
            if r % 16 == 0:
                cv = cidx_v[pl.ds(r, 16)]
            cps.append(pltpu.async_copy(
                ctx_tab.at[cv[r % 16]], crows_v.at[r], sem))
        for cp in cps:
            cp.wait()

        def group_body(g, _):
            for j in range(_G):
                row = g * _G + j
                t = [trows_v[row, pl.ds(16 * m, 16)] for m in range(4)]
                for c in range(C_):
                    rl = j * C_ + c
                    r = g * (_G * C_) + rl
                    acc = crows_v[r, pl.ds(0, 16)] * t[0]
                    acc += crows_v[r, pl.ds(16, 16)] * t[1]
                    acc += crows_v[r, pl.ds(32, 16)] * t[2]
                    acc += crows_v[r, pl.ds(48, 16)] * t[3]
                    plsc.store_scatter(accmat_v, [sidx + rl], acc)
            for k in range(_G * C_ // 16):
                s = accmat_v[pl.ds(16 * k, 16)]
                for m in range(1, 16):
                    s += accmat_v[pl.ds(m * _G * C_ + 16 * k, 16)]
                out_v[pl.ds(g * _G * C_ + 16 * k, 16)] = s
            return _

        lax.fori_loop(0, _CB // _G, group_body, None)
        pltpu.sync_copy(out_v, out_hbm.at[pl.ds(flat, _CB * C_)])
        return _

    lax.fori_loop(0, _NCHUNK, chunk_body, None)


@jax.jit
def _run(target_words, context_flat, tgt_tab, ctx_tab):
    mesh = plsc.VectorSubcoreMesh(core_axis_name="c", subcore_axis_name="s")
    k = functools.partial(
        pl.kernel,
        mesh=mesh,
        compiler_params=pltpu.CompilerParams(needs_layout_passes=False),
        out_type=jax.ShapeDtypeStruct((B_ * C_,), jnp.float32),
        scratch_types=[
            pltpu.VMEM((_CB,), jnp.int32),
            pltpu.VMEM((_CB, EMBED), jnp.float32),
            pltpu.VMEM((_CB * C_,), jnp.int32),
            pltpu.VMEM((_CB * C_, EMBED), jnp.float32),
            pltpu.VMEM((_CB * C_,), jnp.float32),
            pltpu.VMEM((16 * _G * C_,), jnp.float32),
            pltpu.SemaphoreType.DMA,
        ],
    )(_sc_kernel)
    return k(tgt_tab, ctx_tab, target_words, context_flat)


def kernel(target_words, context_words, target_table, context_table):
    context_flat = context_words.reshape(-1)
    return _run(target_words, context_flat, target_table,
                context_table).reshape(B_, C_)


# speedup vs baseline: 1.3929x; 1.0448x over previous
import functools

import jax
import jax.numpy as jnp
from jax import lax
from jax.experimental import pallas as pl
from jax.experimental.pallas import tpu as pltpu
from jax.experimental.pallas import tpu_sc as plsc

VOCAB_ = 1000000
EMBED = 64
B_ = 16384
C_ = 20

_NC = 2
_NS = 16
_NW = _NC * _NS
_BPW = B_ // _NW
_CB = 16
_NCHUNK = _BPW // _CB
_G = 4
_NROW = _CB * C_


def _sc_kernel(tgt_tab, ctx_tab, tidx_hbm, cidx_hbm, out_hbm,
               tcid_a, tcid_b, trows_a, trows_b, cidx_a, cidx_b,
               crows_a, crows_b, out_v, accmat_v, sem_a, sem_b):
    wid = lax.axis_index("s") * _NC + lax.axis_index("c")
    base = wid * _BPW
    lane = lax.broadcasted_iota(jnp.int32, (16,), 0)
    sidx = lane * (_G * C_)

    def fire(i, tcid_v, trows_v, cidx_v, crows_v, sem):
        flat = (base + i * _CB) * C_
        pltpu.sync_copy(tidx_hbm.at[pl.ds(base + i * _CB, _CB)], tcid_v)
        pltpu.sync_copy(cidx_hbm.at[pl.ds(flat, _NROW)], cidx_v)
        tv = tcid_v[pl.ds(0, 16)]
        for j in range(_CB):
            pltpu.async_copy(tgt_tab.at[tv[j]], trows_v.at[j], sem)

        def floop(k, _):
            cv = cidx_v[pl.ds(k * 16, 16)]
            for ln in range(16):
                pltpu.async_copy(
                    ctx_tab.at[cv[ln]], crows_v.at[k * 16 + ln], sem)
            return _

        lax.fori_loop(0, _NROW // 16, floop, None)

    def drain(trows_v, crows_v, sem):
        for j in range(_CB):
            pltpu.make_async_copy(tgt_tab.at[0], trows_v.at[j], sem).wait()

        def dloop(k, _):
            for ln in range(16):
                pltpu.make_async_copy(
                    ctx_tab.at[0], crows_v.at[k * 16 + ln], sem).wait()
            return _

        lax.fori_loop(0, _NROW // 16, dloop, None)

    def compute(i, trows_v, crows_v):
        def group_body(g, _):
            for j in range(_G):
                row = g * _G + j
                t = [trows_v[row, pl.ds(16 * m, 16)] for m in range(4)]
                for c in range(C_):
                    rl = j * C_ + c
                    r = g * (_G * C_) + rl
                    acc = crows_v[r, pl.ds(0, 16)] * t[0]
                    acc += crows_v[r, pl.ds(16, 16)] * t[1]
                    acc += crows_v[r, pl.ds(32, 16)] * t[2]
                    acc += crows_v[r, pl.ds(48, 16)] * t[3]
                    plsc.store_scatter(accmat_v, [sidx + rl], acc)
            for k in range(_G * C_ // 16):
                s = accmat_v[pl.ds(16 * k, 16)]
                for m in range(1, 16):
                    s += accmat_v[pl.ds(m * _G * C_ + 16 * k, 16)]
                out_v[pl.ds(g * _G * C_ + 16 * k, 16)] = s
            return _

        lax.fori_loop(0, _CB // _G, group_body, None)
        pltpu.sync_copy(out_v, out_hbm.at[pl.ds((base + i * _CB) * C_, _NROW)])

    fire(0, tcid_a, trows_a, cidx_a, crows_a, sem_a)
    fire(1, tcid_b, trows_b, cidx_b, crows_b, sem_b)

    def pair_body(it, _):
        i = it * 2
        drain(trows_a, crows_a, sem_a)
        compute(i, trows_a, crows_a)
        fire(i + 2, tcid_a, trows_a, cidx_a, crows_a, sem_a)
        drain(trows_b, crows_b, sem_b)
        compute(i + 1, trows_b, crows_b)
        fire(i + 3, tcid_b, trows_b, cidx_b, crows_b, sem_b)
        return _

    lax.fori_loop(0, _NCHUNK // 2 - 1, pair_body, None)
    i = _NCHUNK - 2
    drain(trows_a, crows_a, sem_a)
    compute(i, trows_a, crows_a)
    drain(trows_b, crows_b, sem_b)
    compute(i + 1, trows_b, crows_b)


@jax.jit
def _run(target_words, context_flat, tgt_tab, ctx_tab):
    mesh = plsc.VectorSubcoreMesh(core_axis_name="c", subcore_axis_name="s")
    k = functools.partial(
        pl.kernel,
        mesh=mesh,
        compiler_params=pltpu.CompilerParams(needs_layout_passes=False),
        out_type=jax.ShapeDtypeStruct((B_ * C_,), jnp.float32),
        scratch_types=[
            pltpu.VMEM((_CB,), jnp.int32),
            pltpu.VMEM((_CB,), jnp.int32),
            pltpu.VMEM((_CB, EMBED), jnp.float32),
            pltpu.VMEM((_CB, EMBED), jnp.float32),
            pltpu.VMEM((_NROW,), jnp.int32),
            pltpu.VMEM((_NROW,), jnp.int32),
            pltpu.VMEM((_NROW, EMBED), jnp.float32),
            pltpu.VMEM((_NROW, EMBED), jnp.float32),
            pltpu.VMEM((_NROW,), jnp.float32),
            pltpu.VMEM((16 * _G * C_,), jnp.float32),
            pltpu.SemaphoreType.DMA,
            pltpu.SemaphoreType.DMA,
        ],
    )(_sc_kernel)
    return k(tgt_tab, ctx_tab, target_words, context_flat)


def kernel(target_words, context_words, target_table, context_table):
    context_flat = context_words.reshape(-1)
    return _run(target_words, context_flat, target_table,
                context_table).reshape(B_, C_)


# speedup vs baseline: 1.3969x; 1.0029x over previous
import functools

import jax
import jax.numpy as jnp
from jax import lax
from jax.experimental import pallas as pl
from jax.experimental.pallas import tpu as pltpu
from jax.experimental.pallas import tpu_sc as plsc
from jax.experimental.layout import Layout, with_layout_constraint

VOCAB_ = 1000000
EMBED = 64
B_ = 16384
C_ = 20

_NC = 2
_NS = 16
_NW = _NC * _NS
_BPW = B_ // _NW
_CB = 16
_NCHUNK = _BPW // _CB
_G = 4
_NROW = _CB * C_


def _sc_kernel(tgt_tab, ctx_tab, tidx_hbm, cidx_hbm, out_hbm,
               tcid_a, tcid_b, trows_a, trows_b, cidx_a, cidx_b,
               crows_a, crows_b, out_v, accmat_v, sem_a, sem_b):
    wid = lax.axis_index("s") * _NC + lax.axis_index("c")
    base = wid * _BPW
    lane = lax.broadcasted_iota(jnp.int32, (16,), 0)
    sidx = lane * (_G * C_)

    def fire(i, tcid_v, trows_v, cidx_v, crows_v, sem):
        flat = (base + i * _CB) * C_
        pltpu.sync_copy(tidx_hbm.at[pl.ds(base + i * _CB, _CB)], tcid_v)
        pltpu.sync_copy(cidx_hbm.at[pl.ds(flat, _NROW)], cidx_v)
        tv = tcid_v[pl.ds(0, 16)]
        for j in range(_CB):
            pltpu.async_copy(tgt_tab.at[tv[j]], trows_v.at[j], sem)

        def floop(k, _):
            cv = cidx_v[pl.ds(k * 16, 16)]
            for ln in range(16):
                pltpu.async_copy(
                    ctx_tab.at[cv[ln]], crows_v.at[k * 16 + ln], sem)
            return _

        lax.fori_loop(0, _NROW // 16, floop, None)

    def drain(trows_v, crows_v, sem):
        for j in range(_CB):
            pltpu.make_async_copy(tgt_tab.at[0], trows_v.at[j], sem).wait()

        def dloop(k, _):
            for ln in range(16):
                pltpu.make_async_copy(
                    ctx_tab.at[0], crows_v.at[k * 16 + ln], sem).wait()
            return _

        lax.fori_loop(0, _NROW // 16, dloop, None)

    def compute(i, trows_v, crows_v):
        def group_body(g, _):
            for j in range(_G):
                row = g * _G + j
                t = [trows_v[row, pl.ds(16 * m, 16)] for m in range(4)]
                for c in range(C_):
                    rl = j * C_ + c
                    r = g * (_G * C_) + rl
                    acc = crows_v[r, pl.ds(0, 16)] * t[0]
                    acc += crows_v[r, pl.ds(16, 16)] * t[1]
                    acc += crows_v[r, pl.ds(32, 16)] * t[2]
                    acc += crows_v[r, pl.ds(48, 16)] * t[3]
                    plsc.store_scatter(accmat_v, [sidx + rl], acc)
            for k in range(_G * C_ // 16):
                s = accmat_v[pl.ds(16 * k, 16)]
                for m in range(1, 16):
                    s += accmat_v[pl.ds(m * _G * C_ + 16 * k, 16)]
                out_v[pl.ds(g * _G * C_ + 16 * k, 16)] = s
            return _

        lax.fori_loop(0, _CB // _G, group_body, None)
        pltpu.sync_copy(out_v, out_hbm.at[pl.ds((base + i * _CB) * C_, _NROW)])

    fire(0, tcid_a, trows_a, cidx_a, crows_a, sem_a)
    fire(1, tcid_b, trows_b, cidx_b, crows_b, sem_b)

    def pair_body(it, _):
        i = it * 2
        drain(trows_a, crows_a, sem_a)
        compute(i, trows_a, crows_a)
        fire(i + 2, tcid_a, trows_a, cidx_a, crows_a, sem_a)
        drain(trows_b, crows_b, sem_b)
        compute(i + 1, trows_b, crows_b)
        fire(i + 3, tcid_b, trows_b, cidx_b, crows_b, sem_b)
        return _

    lax.fori_loop(0, _NCHUNK // 2 - 1, pair_body, None)
    i = _NCHUNK - 2
    drain(trows_a, crows_a, sem_a)
    compute(i, trows_a, crows_a)
    drain(trows_b, crows_b, sem_b)
    compute(i + 1, trows_b, crows_b)


@jax.jit
def _run(target_words, context_flat, tgt_tab, ctx_tab):
    mesh = plsc.VectorSubcoreMesh(core_axis_name="c", subcore_axis_name="s")
    k = functools.partial(
        pl.kernel,
        mesh=mesh,
        compiler_params=pltpu.CompilerParams(needs_layout_passes=False),
        out_type=jax.ShapeDtypeStruct((B_ * C_,), jnp.float32),
        scratch_types=[
            pltpu.VMEM((_CB,), jnp.int32),
            pltpu.VMEM((_CB,), jnp.int32),
            pltpu.VMEM((_CB, EMBED), jnp.float32),
            pltpu.VMEM((_CB, EMBED), jnp.float32),
            pltpu.VMEM((_NROW,), jnp.int32),
            pltpu.VMEM((_NROW,), jnp.int32),
            pltpu.VMEM((_NROW, EMBED), jnp.float32),
            pltpu.VMEM((_NROW, EMBED), jnp.float32),
            pltpu.VMEM((_NROW,), jnp.float32),
            pltpu.VMEM((16 * _G * C_,), jnp.float32),
            pltpu.SemaphoreType.DMA,
            pltpu.SemaphoreType.DMA,
        ],
    )(_sc_kernel)
    return k(tgt_tab, ctx_tab, target_words, context_flat)


def kernel(target_words, context_words, target_table, context_table):
    context_flat = context_words.reshape(-1)
    fmt = Layout(major_to_minor=(1, 0))
    tgt = with_layout_constraint(target_table, fmt)
    ctx = with_layout_constraint(context_table, fmt)
    return _run(target_words, context_flat, tgt, ctx).reshape(B_, C_)
